# baseline (speedup 1.0000x reference)
import functools

import jax
import jax.numpy as jnp
from jax import lax
from jax.experimental import pallas as pl
from jax.experimental.pallas import tpu as pltpu
from jax.experimental.pallas import tpu_sc as plsc

N_NODES = 10000
FEAT = 128
NUM_GRAPHS = 16

NC = 2
NS = 16
NW = NC * NS
CH = 160
SB = 16
NSB = 4
SBE = SB * CH
E_PAD = NW * NSB * SBE
N_PAD = 10240
ROWS_PT = N_PAD // NS


def _mesh():
    return plsc.VectorSubcoreMesh(
        core_axis_name="c", subcore_axis_name="s",
        num_cores=NC, num_subcores=NS)


def _deg_body(eidx_hbm, deg_out, idx_v, ones_v, zrow_v, shared_deg):
    cid = lax.axis_index("c")
    sid = lax.axis_index("s")
    wid = cid * NS + sid

    def _fill1(r, _):
        ones_v[pl.ds(r * 16, 16)] = jnp.full((16,), 1.0, jnp.float32)
        return 0
    lax.fori_loop(0, CH // 16, _fill1, 0)

    def _fill0(r, _):
        zrow_v[pl.ds(r * 16, 16)] = jnp.zeros((16,), jnp.float32)
        return 0
    lax.fori_loop(0, ROWS_PT // 16, _fill0, 0)

    pltpu.sync_copy(zrow_v, shared_deg.at[pl.ds(sid * ROWS_PT, ROWS_PT)])
    plsc.subcore_barrier()

    for t in range(NSB):
        pltpu.sync_copy(eidx_hbm.at[wid].at[t], idx_v)

        def _scat(g, _):
            pltpu.sync_copy(
                ones_v, shared_deg.at[idx_v.at[pl.ds(SBE + g * CH, CH)]],
                add=True)
            return 0
        lax.fori_loop(0, SB, _scat, 0)

    plsc.subcore_barrier()
    pltpu.sync_copy(shared_deg.at[pl.ds(sid * ROWS_PT, ROWS_PT)],
                    deg_out.at[cid].at[pl.ds(sid * ROWS_PT, ROWS_PT)])


@functools.cache
def _deg_call():
    return pl.kernel(
        _deg_body,
        out_type=jax.ShapeDtypeStruct((NC, N_PAD), jnp.float32),
        mesh=_mesh(),
        scratch_types=[
            pltpu.VMEM((2 * SBE,), jnp.int32),
            pltpu.VMEM((CH,), jnp.float32),
            pltpu.VMEM((ROWS_PT,), jnp.float32),
            pltpu.VMEM_SHARED((N_PAD,), jnp.float32),
        ],
    )


def _agg_body(u_hbm, eidx_hbm, agg_out,
              idx_v, rows0, rows1, shared_agg, sem0, sem1):
    cid = lax.axis_index("c")
    sid = lax.axis_index("s")
    wid = cid * NS + sid

    def _zfill(r, _):
        for c in range(FEAT // 16):
            rows0[r, pl.ds(c * 16, 16)] = jnp.zeros((16,), jnp.float32)
        return 0
    lax.fori_loop(0, CH, _zfill, 0)
    for k in range(ROWS_PT // CH):
        pltpu.sync_copy(
            rows0, shared_agg.at[pl.ds(sid * ROWS_PT + k * CH, CH)])
    plsc.subcore_barrier()

    def _gather(g, rows, sem):
        pltpu.async_copy(u_hbm.at[idx_v.at[pl.ds(g * CH, CH)]], rows, sem)

    def _wait_g(sem, rows):
        pltpu.make_async_copy(
            u_hbm.at[idx_v.at[pl.ds(0, CH)]], rows, sem).wait()

    def _scat(g, rows):
        pltpu.sync_copy(
            rows, shared_agg.at[idx_v.at[pl.ds(SBE + g * CH, CH)]],
            add=True)

    for t in range(NSB):
        pltpu.sync_copy(eidx_hbm.at[wid].at[t], idx_v)
        _gather(0, rows0, sem0)
        _gather(1, rows1, sem1)

        def _step(i, _):
            g0 = 2 * i
            _wait_g(sem0, rows0)
            _scat(g0, rows0)
            _gather(g0 + 2, rows0, sem0)
            _wait_g(sem1, rows1)
            _scat(g0 + 1, rows1)
            _gather(g0 + 3, rows1, sem1)
            return 0
        lax.fori_loop(0, SB // 2 - 1, _step, 0)

        _wait_g(sem0, rows0)
        _scat(SB - 2, rows0)
        _wait_g(sem1, rows1)
        _scat(SB - 1, rows1)

    plsc.subcore_barrier()
    pltpu.sync_copy(shared_agg.at[pl.ds(sid * ROWS_PT, ROWS_PT)],
                    agg_out.at[cid].at[pl.ds(sid * ROWS_PT, ROWS_PT)])


@functools.cache
def _agg_call():
    return pl.kernel(
        _agg_body,
        out_type=jax.ShapeDtypeStruct((NC, N_PAD, FEAT), jnp.float32),
        mesh=_mesh(),
        scratch_types=[
            pltpu.VMEM((2 * SBE,), jnp.int32),
            pltpu.VMEM((CH, FEAT), jnp.float32),
            pltpu.VMEM((CH, FEAT), jnp.float32),
            pltpu.VMEM_SHARED((N_PAD, FEAT), jnp.float32),
            pltpu.SemaphoreType.DMA,
            pltpu.SemaphoreType.DMA,
        ],
    )


def _prep_body(degT_ref, x_ref, dinv_ref, u0_ref):
    deg = degT_ref[:, 0:1] + degT_ref[:, 1:2] + 1.0
    rows = lax.broadcasted_iota(jnp.int32, (N_PAD, 1), 0)
    dinv = jnp.where(rows < N_NODES, lax.rsqrt(deg), 0.0)
    dinv_ref[...] = dinv
    u0_ref[...] = x_ref[...] * dinv


def _prep_call(degT, x_pad):
    return pl.pallas_call(
        _prep_body,
        out_shape=[
            jax.ShapeDtypeStruct((N_PAD, 1), jnp.float32),
            jax.ShapeDtypeStruct((N_PAD, FEAT), jnp.float32),
        ],
    )(degT, x_pad)


def _block_core(prev, h, aggp_ref, dinv, W, b, gamma, beta):
    agg = aggp_ref[0] + aggp_ref[1]
    s = dinv * agg + (dinv * dinv) * h
    conv = jnp.dot(s, W, preferred_element_type=jnp.float32) + b
    z = prev + conv
    rows = lax.broadcasted_iota(jnp.int32, (N_PAD, 1), 0)
    mask = rows < N_NODES
    z = jnp.where(mask, z, 0.0)
    mean = jnp.sum(z, axis=0, keepdims=True) / N_NODES
    cz = jnp.where(mask, z - mean, 0.0)
    var = jnp.sum(cz * cz, axis=0, keepdims=True) / N_NODES
    zn = cz * lax.rsqrt(var + 1e-5) * gamma + beta
    out = jnp.where(zn > 0, zn, jnp.exp(zn) - 1.0)
    return jnp.where(mask, out, 0.0)


def _block_body(prev_ref, h_ref, aggp_ref, dinv_ref, W_ref, b_ref,
                g_ref, be_ref, h_out, u_out):
    dinv = dinv_ref[...]
    hn = _block_core(prev_ref[...], h_ref[...], aggp_ref, dinv,
                     W_ref[...], b_ref[...], g_ref[...], be_ref[...])
    h_out[...] = hn
    u_out[...] = hn * dinv


def _block_call(prev, h, aggp, dinv, W, b, gamma, beta):
    return pl.pallas_call(
        _block_body,
        out_shape=[
            jax.ShapeDtypeStruct((N_PAD, FEAT), jnp.float32),
            jax.ShapeDtypeStruct((N_PAD, FEAT), jnp.float32),
        ],
    )(prev, h, aggp, dinv, W, b, gamma, beta)


def _final_body(prev_ref, h_ref, aggp_ref, dinv_ref, W_ref, b_ref,
                g_ref, be_ref, batch_ref, Wr_ref, br_ref, out_ref):
    hn = _block_core(prev_ref[...], h_ref[...], aggp_ref, dinv_ref[...],
                     W_ref[...], b_ref[...], g_ref[...], be_ref[...])
    rows = lax.broadcasted_iota(jnp.int32, (N_PAD, 1), 0)
    gids = lax.broadcasted_iota(jnp.int32, (1, NUM_GRAPHS), 1)
    M = jnp.where((batch_ref[...] == gids) & (rows < N_NODES), 1.0, 0.0)
    sums = lax.dot_general(M, hn, (((0,), (0,)), ((), ())),
                           preferred_element_type=jnp.float32)
    ones_col = jnp.where(rows < N_NODES, 1.0, 0.0)
    counts = lax.dot_general(M, ones_col, (((0,), (0,)), ((), ())),
                             preferred_element_type=jnp.float32)
    pooled = sums / jnp.maximum(counts, 1.0)
    out_ref[...] = (jnp.dot(pooled, Wr_ref[...],
                            preferred_element_type=jnp.float32) + br_ref[...])


def _final_call(prev, h, aggp, dinv, batch2d, W, b, gamma, beta, Wr_pad, br_pad):
    return pl.pallas_call(
        _final_body,
        out_shape=jax.ShapeDtypeStruct((NUM_GRAPHS, FEAT), jnp.float32),
    )(prev, h, aggp, dinv, W, b, gamma, beta, batch2d, Wr_pad, br_pad)


def kernel(x, edge_index, batch, Ws, bs, gammas, betas, Wr, br):
    n_edges = edge_index.shape[1]
    pad_e = E_PAD - n_edges
    pad_idx = N_NODES + (jnp.arange(pad_e, dtype=jnp.int32) % (N_PAD - N_NODES))
    src = jnp.concatenate([edge_index[0], pad_idx]).reshape(NW, NSB, 1, SBE)
    dst = jnp.concatenate([edge_index[1], pad_idx]).reshape(NW, NSB, 1, SBE)
    eidx = jnp.concatenate([src, dst], axis=2).reshape(NW, NSB, 2 * SBE)

    x_pad = jnp.zeros((N_PAD, FEAT), x.dtype).at[:N_NODES].set(x)
    batch2d = jnp.full((N_PAD, 1), NUM_GRAPHS + 1, jnp.int32).at[:N_NODES, 0].set(batch)
    Wr_pad = jnp.zeros((FEAT, FEAT), Wr.dtype).at[:, :Wr.shape[1]].set(Wr)
    br_pad = jnp.zeros((1, FEAT), br.dtype).at[0, :br.shape[0]].set(br)

    deg_p = _deg_call()(eidx)
    dinv, u = _prep_call(deg_p.T, x_pad)

    h = x_pad
    prev = jnp.zeros_like(x_pad)
    for i in range(Ws.shape[0] - 1):
        aggp = _agg_call()(u, eidx)
        h_new, u = _block_call(prev, h, aggp, dinv, Ws[i],
                               bs[i][None, :], gammas[i][None, :],
                               betas[i][None, :])
        prev, h = h, h_new

    i = Ws.shape[0] - 1
    aggp = _agg_call()(u, eidx)
    out = _final_call(prev, h, aggp, dinv, batch2d, Ws[i], bs[i][None, :],
                      gammas[i][None, :], betas[i][None, :], Wr_pad, br_pad)
    return out[:, :Wr.shape[1]]


# speedup vs baseline: 1.0351x; 1.0351x over previous
import functools

import jax
import jax.numpy as jnp
from jax import lax
from jax.experimental import pallas as pl
from jax.experimental.pallas import tpu as pltpu
from jax.experimental.pallas import tpu_sc as plsc

N_NODES = 10000
FEAT = 128
NUM_GRAPHS = 16

NC = 2
NS = 16
NW = NC * NS
CH = 128
SB = 20
NSB = 4
SBE = SB * CH
E_PAD = NW * NSB * SBE
N_PAD = 10240
ROWS_PT = N_PAD // NS


def _mesh():
    return plsc.VectorSubcoreMesh(
        core_axis_name="c", subcore_axis_name="s",
        num_cores=NC, num_subcores=NS)


def _deg_body(eidx_hbm, deg_out, idx_v, ones_v, zrow_v, shared_deg):
    cid = lax.axis_index("c")
    sid = lax.axis_index("s")
    wid = cid * NS + sid

    def _fill1(r, _):
        ones_v[pl.ds(r * 16, 16)] = jnp.full((16,), 1.0, jnp.float32)
        return 0
    lax.fori_loop(0, CH // 16, _fill1, 0)

    def _fill0(r, _):
        zrow_v[pl.ds(r * 16, 16)] = jnp.zeros((16,), jnp.float32)
        return 0
    lax.fori_loop(0, ROWS_PT // 16, _fill0, 0)

    pltpu.sync_copy(zrow_v, shared_deg.at[pl.ds(sid * ROWS_PT, ROWS_PT)])
    plsc.subcore_barrier()

    for t in range(NSB):
        pltpu.sync_copy(eidx_hbm.at[wid].at[t], idx_v)

        def _scat(g, _):
            pltpu.sync_copy(
                ones_v, shared_deg.at[idx_v.at[pl.ds(SBE + g * CH, CH)]],
                add=True)
            return 0
        lax.fori_loop(0, SB, _scat, 0)

    plsc.subcore_barrier()
    pltpu.sync_copy(shared_deg.at[pl.ds(sid * ROWS_PT, ROWS_PT)],
                    deg_out.at[cid].at[pl.ds(sid * ROWS_PT, ROWS_PT)])


@functools.cache
def _deg_call():
    return pl.kernel(
        _deg_body,
        out_type=jax.ShapeDtypeStruct((NC, N_PAD), jnp.float32),
        mesh=_mesh(),
        scratch_types=[
            pltpu.VMEM((2 * SBE,), jnp.int32),
            pltpu.VMEM((CH,), jnp.float32),
            pltpu.VMEM((ROWS_PT,), jnp.float32),
            pltpu.VMEM_SHARED((N_PAD,), jnp.float32),
        ],
    )


def _agg_body(u_hbm, eidx_hbm, agg_out,
              idxA, idxB, rows0, rows1, shared_agg, sem0, sem1, isem):
    cid = lax.axis_index("c")
    sid = lax.axis_index("s")
    wid = cid * NS + sid

    def _zfill(r, _):
        for c in range(FEAT // 16):
            rows0[r, pl.ds(c * 16, 16)] = jnp.zeros((16,), jnp.float32)
        return 0
    lax.fori_loop(0, CH, _zfill, 0)
    for k in range(ROWS_PT // CH):
        pltpu.sync_copy(
            rows0, shared_agg.at[pl.ds(sid * ROWS_PT + k * CH, CH)])
    plsc.subcore_barrier()

    def _gather(idx_v, g, rows, sem):
        pltpu.async_copy(u_hbm.at[idx_v.at[pl.ds(g * CH, CH)]], rows, sem)

    def _wait_g(sem, rows):
        pltpu.make_async_copy(
            u_hbm.at[idxA.at[pl.ds(0, CH)]], rows, sem).wait()

    def _scat(idx_v, g, rows):
        pltpu.sync_copy(
            rows, shared_agg.at[idx_v.at[pl.ds(SBE + g * CH, CH)]],
            add=True)

    pltpu.sync_copy(eidx_hbm.at[wid].at[0], idxA)
    pltpu.async_copy(eidx_hbm.at[wid].at[1], idxB, isem)
    _gather(idxA, 0, rows0, sem0)
    _gather(idxA, 1, rows1, sem1)

    for t in range(NSB):
        cur, nxt = (idxA, idxB) if t % 2 == 0 else (idxB, idxA)

        def _step(i, _):
            g0 = 2 * i
            _wait_g(sem0, rows0)
            _scat(cur, g0, rows0)
            _gather(cur, g0 + 2, rows0, sem0)
            g1 = g0 + 1
            _wait_g(sem1, rows1)
            _scat(cur, g1, rows1)
            _gather(cur, g1 + 2, rows1, sem1)
            return 0
        lax.fori_loop(0, SB // 2 - 2, _step, 0)

        _wait_g(sem0, rows0)
        _scat(cur, SB - 4, rows0)
        _gather(cur, SB - 2, rows0, sem0)
        _wait_g(sem1, rows1)
        _scat(cur, SB - 3, rows1)
        _gather(cur, SB - 1, rows1, sem1)

        if t < NSB - 1:
            pltpu.make_async_copy(eidx_hbm.at[wid].at[t], nxt, isem).wait()

        _wait_g(sem0, rows0)
        _scat(cur, SB - 2, rows0)
        if t < NSB - 1:
            _gather(nxt, 0, rows0, sem0)
        _wait_g(sem1, rows1)
        _scat(cur, SB - 1, rows1)
        if t < NSB - 1:
            _gather(nxt, 1, rows1, sem1)

        if t < NSB - 2:
            pltpu.async_copy(eidx_hbm.at[wid].at[t + 2], cur, isem)

    plsc.subcore_barrier()
    pltpu.sync_copy(shared_agg.at[pl.ds(sid * ROWS_PT, ROWS_PT)],
                    agg_out.at[cid].at[pl.ds(sid * ROWS_PT, ROWS_PT)])


@functools.cache
def _agg_call():
    return pl.kernel(
        _agg_body,
        out_type=jax.ShapeDtypeStruct((NC, N_PAD, FEAT), jnp.float32),
        mesh=_mesh(),
        scratch_types=[
            pltpu.VMEM((2 * SBE,), jnp.int32),
            pltpu.VMEM((2 * SBE,), jnp.int32),
            pltpu.VMEM((CH, FEAT), jnp.float32),
            pltpu.VMEM((CH, FEAT), jnp.float32),
            pltpu.VMEM_SHARED((N_PAD, FEAT), jnp.float32),
            pltpu.SemaphoreType.DMA,
            pltpu.SemaphoreType.DMA,
            pltpu.SemaphoreType.DMA,
        ],
    )


def _prep_body(degT_ref, x_ref, dinv_ref, u0_ref):
    deg = degT_ref[:, 0:1] + degT_ref[:, 1:2] + 1.0
    rows = lax.broadcasted_iota(jnp.int32, (N_PAD, 1), 0)
    dinv = jnp.where(rows < N_NODES, lax.rsqrt(deg), 0.0)
    dinv_ref[...] = dinv
    u0_ref[...] = x_ref[...] * dinv


def _prep_call(degT, x_pad):
    return pl.pallas_call(
        _prep_body,
        out_shape=[
            jax.ShapeDtypeStruct((N_PAD, 1), jnp.float32),
            jax.ShapeDtypeStruct((N_PAD, FEAT), jnp.float32),
        ],
    )(degT, x_pad)


def _block_core(prev, h, aggp_ref, dinv, W, b, gamma, beta):
    agg = aggp_ref[0] + aggp_ref[1]
    s = dinv * agg + (dinv * dinv) * h
    conv = jnp.dot(s, W, preferred_element_type=jnp.float32) + b
    z = prev + conv
    rows = lax.broadcasted_iota(jnp.int32, (N_PAD, 1), 0)
    mask = rows < N_NODES
    z = jnp.where(mask, z, 0.0)
    mean = jnp.sum(z, axis=0, keepdims=True) / N_NODES
    cz = jnp.where(mask, z - mean, 0.0)
    var = jnp.sum(cz * cz, axis=0, keepdims=True) / N_NODES
    zn = cz * lax.rsqrt(var + 1e-5) * gamma + beta
    out = jnp.where(zn > 0, zn, jnp.exp(zn) - 1.0)
    return jnp.where(mask, out, 0.0)


def _block_body(prev_ref, h_ref, aggp_ref, dinv_ref, W_ref, b_ref,
                g_ref, be_ref, h_out, u_out):
    dinv = dinv_ref[...]
    hn = _block_core(prev_ref[...], h_ref[...], aggp_ref, dinv,
                     W_ref[...], b_ref[...], g_ref[...], be_ref[...])
    h_out[...] = hn
    u_out[...] = hn * dinv


def _block_call(prev, h, aggp, dinv, W, b, gamma, beta):
    return pl.pallas_call(
        _block_body,
        out_shape=[
            jax.ShapeDtypeStruct((N_PAD, FEAT), jnp.float32),
            jax.ShapeDtypeStruct((N_PAD, FEAT), jnp.float32),
        ],
    )(prev, h, aggp, dinv, W, b, gamma, beta)


def _final_body(prev_ref, h_ref, aggp_ref, dinv_ref, W_ref, b_ref,
                g_ref, be_ref, batch_ref, Wr_ref, br_ref, out_ref):
    hn = _block_core(prev_ref[...], h_ref[...], aggp_ref, dinv_ref[...],
                     W_ref[...], b_ref[...], g_ref[...], be_ref[...])
    rows = lax.broadcasted_iota(jnp.int32, (N_PAD, 1), 0)
    gids = lax.broadcasted_iota(jnp.int32, (1, NUM_GRAPHS), 1)
    M = jnp.where((batch_ref[...] == gids) & (rows < N_NODES), 1.0, 0.0)
    sums = lax.dot_general(M, hn, (((0,), (0,)), ((), ())),
                           preferred_element_type=jnp.float32)
    ones_col = jnp.where(rows < N_NODES, 1.0, 0.0)
    counts = lax.dot_general(M, ones_col, (((0,), (0,)), ((), ())),
                             preferred_element_type=jnp.float32)
    pooled = sums / jnp.maximum(counts, 1.0)
    out_ref[...] = (jnp.dot(pooled, Wr_ref[...],
                            preferred_element_type=jnp.float32) + br_ref[...])


def _final_call(prev, h, aggp, dinv, batch2d, W, b, gamma, beta, Wr_pad, br_pad):
    return pl.pallas_call(
        _final_body,
        out_shape=jax.ShapeDtypeStruct((NUM_GRAPHS, FEAT), jnp.float32),
    )(prev, h, aggp, dinv, W, b, gamma, beta, batch2d, Wr_pad, br_pad)


def kernel(x, edge_index, batch, Ws, bs, gammas, betas, Wr, br):
    n_edges = edge_index.shape[1]
    pad_e = E_PAD - n_edges
    pad_idx = N_NODES + (jnp.arange(pad_e, dtype=jnp.int32) % (N_PAD - N_NODES))
    src = jnp.concatenate([edge_index[0], pad_idx]).reshape(NW, NSB, 1, SBE)
    dst = jnp.concatenate([edge_index[1], pad_idx]).reshape(NW, NSB, 1, SBE)
    eidx = jnp.concatenate([src, dst], axis=2).reshape(NW, NSB, 2 * SBE)

    x_pad = jnp.zeros((N_PAD, FEAT), x.dtype).at[:N_NODES].set(x)
    batch2d = jnp.full((N_PAD, 1), NUM_GRAPHS + 1, jnp.int32).at[:N_NODES, 0].set(batch)
    Wr_pad = jnp.zeros((FEAT, FEAT), Wr.dtype).at[:, :Wr.shape[1]].set(Wr)
    br_pad = jnp.zeros((1, FEAT), br.dtype).at[0, :br.shape[0]].set(br)

    deg_p = _deg_call()(eidx)
    dinv, u = _prep_call(deg_p.T, x_pad)

    h = x_pad
    prev = jnp.zeros_like(x_pad)
    for i in range(Ws.shape[0] - 1):
        aggp = _agg_call()(u, eidx)
        h_new, u = _block_call(prev, h, aggp, dinv, Ws[i],
                               bs[i][None, :], gammas[i][None, :],
                               betas[i][None, :])
        prev, h = h, h_new

    i = Ws.shape[0] - 1
    aggp = _agg_call()(u, eidx)
    out = _final_call(prev, h, aggp, dinv, batch2d, Ws[i], bs[i][None, :],
                      gammas[i][None, :], betas[i][None, :], Wr_pad, br_pad)
    return out[:, :Wr.shape[1]]
